# baseline (speedup 1.0000x reference)
import functools

import jax
import jax.numpy as jnp
from jax import lax
from jax.experimental import pallas as pl
from jax.experimental.pallas import tpu as pltpu
from jax.experimental.pallas import tpu_sc as plsc

N = 10000
NP = 10240
E = 320000
D = 128
NC = 2
NS = 16
NW = NC * NS
E_PER_W = E // NW
CH = 128
NCH = 79
EPW = NCH * CH
RPS = NP // NS
ZR = 8

_MESH = dict(core_axis_name="c", subcore_axis_name="s",
             num_cores=NC, num_subcores=NS)


def _fill(ref, rows, value):
    @pl.loop(0, rows)
    def _(r):
        @pl.loop(0, D // 16)
        def _(c):
            ref[r, pl.ds(c * 16, 16)] = jnp.full((16,), value, jnp.float32)


def _zero_accum(sid, zrow_v, accum_sh):
    @pl.loop(0, RPS // ZR)
    def _(j):
        pltpu.sync_copy(zrow_v, accum_sh.at[pl.ds(sid * RPS + j * ZR, ZR), :])


def _sc_agg_body(with_deg, h_hbm, srcf_hbm, dstf_hbm, *refs):
    if with_deg:
        (out_hbm, deg_out, sidx_v, didx_v, rows0_v, rows1_v,
         zrow_v, accum_sh, *sems) = refs
    else:
        (out_hbm, sidx_v, didx_v, rows0_v, rows1_v,
         zrow_v, accum_sh, *sems) = refs
    cid = lax.axis_index("c")
    sid = lax.axis_index("s")
    wid = sid * NC + cid
    ebase = wid * EPW
    rs = pl.ds(sid * RPS, RPS)

    isems = sems[0:4]
    rsems = sems[4:6]
    rows = (rows0_v, rows1_v)

    _fill(zrow_v, ZR, 0.0)
    _zero_accum(sid, zrow_v, accum_sh)
    if with_deg:
        _fill(rows0_v, CH, 1.0)
    plsc.subcore_barrier()

    def idx_load(m, s, need_src=True):
        off = pl.multiple_of(ebase + m * CH, 8)
        if need_src:
            pltpu.async_copy(srcf_hbm.at[pl.ds(off, CH)], sidx_v.at[s],
                             isems[s])
        pltpu.async_copy(dstf_hbm.at[pl.ds(off, CH)], didx_v.at[s], isems[s])

    def idx_wait(s, need_src=True):
        if need_src:
            pltpu.make_async_copy(srcf_hbm.at[pl.ds(0, CH)], sidx_v.at[s],
                                  isems[s]).wait()
        pltpu.make_async_copy(dstf_hbm.at[pl.ds(0, CH)], didx_v.at[s],
                              isems[s]).wait()

    def gather_start(s, r):
        pltpu.async_copy(h_hbm.at[sidx_v.at[s]], rows[r], rsems[r])

    def gather_wait(r):
        pltpu.make_async_copy(h_hbm.at[sidx_v.at[0]], rows[r], rsems[r]).wait()

    def scatter_sync(s, r):
        pltpu.sync_copy(rows[r], accum_sh.at[didx_v.at[s]], add=True)

    if with_deg:
        def deg_body(m, b, dyn):
            idx_wait(b, need_src=False)
            scatter_sync(b, 0)
            if dyn:
                @pl.when(m + 4 < NCH)
                def _():
                    idx_load(m + 4, b, need_src=False)
            elif m + 4 < NCH:
                idx_load(m + 4, b, need_src=False)

        for s in range(4):
            idx_load(s, s, need_src=False)
        for m in range(4):
            deg_body(m, m % 4, False)

        @pl.loop(1, NCH // 4)
        def _(q):
            for b in range(4):
                deg_body(q * 4 + b, b, True)

        for m in range((NCH // 4) * 4, NCH):
            deg_body(m, m % 4, False)

        plsc.subcore_barrier()
        pltpu.sync_copy(accum_sh.at[rs, :], deg_out.at[cid, rs, :])
        _zero_accum(sid, zrow_v, accum_sh)
        plsc.subcore_barrier()

    def agg_body(m, b, dyn):
        gather_wait(b % 2)
        scatter_sync(b, b % 2)
        if dyn:
            @pl.when(m + 4 < NCH)
            def _():
                idx_load(m + 4, b)

            @pl.when(m + 2 < NCH)
            def _():
                idx_wait((b + 2) % 4)
                gather_start((b + 2) % 4, b % 2)
        else:
            if m + 4 < NCH:
                idx_load(m + 4, b)
            if m + 2 < NCH:
                idx_wait((b + 2) % 4)
                gather_start((b + 2) % 4, b % 2)

    for s in range(4):
        idx_load(s, s)
    idx_wait(0)
    gather_start(0, 0)
    idx_wait(1)
    gather_start(1, 1)

    for m in range(4):
        agg_body(m, m % 4, False)

    @pl.loop(1, NCH // 4)
    def _(q):
        for b in range(4):
            agg_body(q * 4 + b, b, True)

    for m in range((NCH // 4) * 4, NCH):
        agg_body(m, m % 4, False)

    plsc.subcore_barrier()

    pltpu.sync_copy(accum_sh.at[rs, :], out_hbm.at[cid, rs, :])


def _make_sc_agg(with_deg):
    out_type = jax.ShapeDtypeStruct((NC, NP, D), jnp.float32)
    return pl.kernel(
        functools.partial(_sc_agg_body, with_deg),
        out_type=(out_type, out_type) if with_deg else out_type,
        mesh=plsc.VectorSubcoreMesh(**_MESH),
        scratch_types=[
            pltpu.VMEM((4, CH), jnp.int32),
            pltpu.VMEM((4, CH), jnp.int32),
            pltpu.VMEM((CH, D), jnp.float32),
            pltpu.VMEM((CH, D), jnp.float32),
            pltpu.VMEM((ZR, D), jnp.float32),
            pltpu.VMEM_SHARED((NP, D), jnp.float32),
        ] + [pltpu.SemaphoreType.DMA] * 6)


def _dense_body(relu, h_ref, aggp_ref, degp_ref, ws_ref, wn_ref, b_ref, o_ref):
    p = aggp_ref[0] + aggp_ref[1]
    deg = degp_ref[0, :, 0:1] + degp_ref[1, :, 0:1]
    hn = p * (1.0 / jnp.maximum(deg, 1.0))
    acc = jnp.dot(h_ref[...], ws_ref[...], preferred_element_type=jnp.float32)
    acc = acc + jnp.dot(hn, wn_ref[...], preferred_element_type=jnp.float32)
    acc = acc + b_ref[...]
    if relu:
        acc = jnp.maximum(acc, 0.0)
    o_ref[...] = acc


BLK = 1000


def _dense(h, aggp, degp, w_self, w_neigh, b, relu):
    return pl.pallas_call(
        functools.partial(_dense_body, relu),
        grid=(N // BLK,),
        in_specs=[
            pl.BlockSpec((BLK, D), lambda i: (i, 0)),
            pl.BlockSpec((NC, BLK, D), lambda i: (0, i, 0)),
            pl.BlockSpec((NC, BLK, D), lambda i: (0, i, 0)),
            pl.BlockSpec((D, D), lambda i: (0, 0)),
            pl.BlockSpec((D, D), lambda i: (0, 0)),
            pl.BlockSpec((1, D), lambda i: (0, 0)),
        ],
        out_specs=pl.BlockSpec((BLK, D), lambda i: (i, 0)),
        out_shape=jax.ShapeDtypeStruct((N, D), jnp.float32),
    )(h, aggp, degp, w_self, w_neigh, b)


def kernel(x, edge_index, W1_self, W1_neigh, b1, W2_self, W2_neigh, b2):
    pad = EPW - E_PER_W
    srcf = jnp.pad(edge_index[0].reshape(NW, E_PER_W),
                   ((0, 0), (0, pad))).reshape(-1)
    dstf = jnp.pad(edge_index[1].reshape(NW, E_PER_W),
                   ((0, 0), (0, pad)), constant_values=N).reshape(-1)
    b1r = b1.reshape(1, D)
    b2r = b2.reshape(1, D)

    agg1p, degp = _make_sc_agg(True)(x, srcf, dstf)
    h1 = _dense(x, agg1p, degp, W1_self, W1_neigh, b1r, relu=True)
    agg2p = _make_sc_agg(False)(h1, srcf, dstf)
    out = _dense(h1, agg2p, degp, W2_self, W2_neigh, b2r, relu=False)
    return out


# speedup vs baseline: 1.7060x; 1.7060x over previous
import functools

import jax
import jax.numpy as jnp
from jax import lax
from jax.experimental import pallas as pl
from jax.experimental.pallas import tpu as pltpu
from jax.experimental.pallas import tpu_sc as plsc

N = 10000
NP = 10240
E = 320000
D = 128
NC = 2
NS = 16
NW = NC * NS
E_PER_W = E // NW
CH = 80
NCH = E_PER_W // CH
RPS = NP // NS
ZR = 8

_MESH = dict(core_axis_name="c", subcore_axis_name="s",
             num_cores=NC, num_subcores=NS)


def _fill(ref, rows, value):
    @pl.loop(0, rows)
    def _(r):
        @pl.loop(0, D // 16)
        def _(c):
            ref[r, pl.ds(c * 16, 16)] = jnp.full((16,), value, jnp.float32)


def _zero_accum(sid, zrow_v, accum_sh):
    @pl.loop(0, RPS // ZR)
    def _(j):
        pltpu.sync_copy(zrow_v, accum_sh.at[pl.ds(sid * RPS + j * ZR, ZR), :])


def _sc_agg_body(with_deg, h_hbm, srcf_hbm, dstf_hbm, *refs):
    if with_deg:
        (out_hbm, deg_out,
         sidx_v, didx_v, rows0_v, rows1_v, rows2_v, rows3_v,
         zrow_v, accum_sh, *sems) = refs
    else:
        (out_hbm,
         sidx_v, didx_v, rows0_v, rows1_v, rows2_v, rows3_v,
         zrow_v, accum_sh, *sems) = refs
    cid = lax.axis_index("c")
    sid = lax.axis_index("s")
    wid = sid * NC + cid
    ebase = wid * E_PER_W
    rs = pl.ds(sid * RPS, RPS)

    isems = sems[0:8]
    rsems = sems[8:12]
    ssems = sems[12:16]
    rows = (rows0_v, rows1_v, rows2_v, rows3_v)

    _fill(zrow_v, ZR, 0.0)
    _zero_accum(sid, zrow_v, accum_sh)
    if with_deg:
        _fill(rows0_v, CH, 1.0)
    plsc.subcore_barrier()

    def idx_load(m, s):
        off = pl.multiple_of(ebase + m * CH, 8)
        pltpu.async_copy(srcf_hbm.at[pl.ds(off, CH)], sidx_v.at[s], isems[s])
        pltpu.async_copy(dstf_hbm.at[pl.ds(off, CH)], didx_v.at[s], isems[s])

    def idx_wait(s):
        pltpu.make_async_copy(srcf_hbm.at[pl.ds(0, CH)], sidx_v.at[s],
                              isems[s]).wait()
        pltpu.make_async_copy(dstf_hbm.at[pl.ds(0, CH)], didx_v.at[s],
                              isems[s]).wait()

    def gather_start(s, r):
        pltpu.async_copy(h_hbm.at[sidx_v.at[s]], rows[r], rsems[r])

    def gather_wait(r):
        pltpu.make_async_copy(h_hbm.at[sidx_v.at[0]], rows[r], rsems[r]).wait()

    def scatter_start(s, r, buf=None):
        pltpu.async_copy(rows[r if buf is None else buf],
                         accum_sh.at[didx_v.at[s]], ssems[r], add=True)

    def scatter_wait(r):
        pltpu.make_async_copy(rows[0], accum_sh.at[didx_v.at[0]],
                              ssems[r]).wait()

    if with_deg:
        def deg_body(m, b8, dyn):
            b4 = b8 % 4
            if dyn:
                scatter_wait(b4)

                @pl.when(m + 4 < NCH)
                def _():
                    idx_load(m + 4, (b8 + 4) % 8)
            else:
                if m >= 4:
                    scatter_wait(b4)
                    if m + 4 < NCH:
                        idx_load(m + 4, (b8 + 4) % 8)
            idx_wait(b8)
            scatter_start(b8, b4, buf=0)

        for s in range(8):
            idx_load(s, s)
        for m in range(8):
            deg_body(m, m % 8, False)

        @pl.loop(1, NCH // 8)
        def _(q):
            for b in range(8):
                deg_body(q * 8 + b, b, True)

        for m in range((NCH // 8) * 8, NCH):
            deg_body(m, m % 8, False)
        for k in range(NCH - 4, NCH):
            scatter_wait(k % 4)

        plsc.subcore_barrier()
        pltpu.sync_copy(accum_sh.at[rs, :], deg_out.at[cid, rs, :])
        _zero_accum(sid, zrow_v, accum_sh)
        plsc.subcore_barrier()

    def agg_body(m, b8, dyn):
        b4 = b8 % 4
        gather_wait(b4)
        scatter_start(b8, b4)
        if dyn or m >= 2:
            scatter_wait((b4 + 2) % 4)
            if dyn:
                @pl.when(m + 6 < NCH)
                def _():
                    idx_load(m + 6, (b8 + 6) % 8)
            elif m + 6 < NCH:
                idx_load(m + 6, (b8 + 6) % 8)
        if dyn:
            @pl.when(m + 2 < NCH)
            def _():
                idx_wait((b8 + 2) % 8)
                gather_start((b8 + 2) % 8, (b4 + 2) % 4)
        elif m + 2 < NCH:
            idx_wait((b8 + 2) % 8)
            gather_start((b8 + 2) % 8, (b4 + 2) % 4)

    for s in range(8):
        idx_load(s, s)
    idx_wait(0)
    gather_start(0, 0)
    idx_wait(1)
    gather_start(1, 1)

    for m in range(8):
        agg_body(m, m % 8, False)

    @pl.loop(1, NCH // 8)
    def _(q):
        for b in range(8):
            agg_body(q * 8 + b, b, True)

    for m in range((NCH // 8) * 8, NCH):
        agg_body(m, m % 8, False)

    scatter_wait((NCH - 2) % 4)
    scatter_wait((NCH - 1) % 4)

    plsc.subcore_barrier()

    pltpu.sync_copy(accum_sh.at[rs, :], out_hbm.at[cid, rs, :])


def _make_sc_agg(with_deg):
    out_type = jax.ShapeDtypeStruct((NC, NP, D), jnp.float32)
    return pl.kernel(
        functools.partial(_sc_agg_body, with_deg),
        out_type=(out_type, out_type) if with_deg else out_type,
        mesh=plsc.VectorSubcoreMesh(**_MESH),
        scratch_types=[
            pltpu.VMEM((8, CH), jnp.int32),
            pltpu.VMEM((8, CH), jnp.int32),
            pltpu.VMEM((CH, D), jnp.float32),
            pltpu.VMEM((CH, D), jnp.float32),
            pltpu.VMEM((CH, D), jnp.float32),
            pltpu.VMEM((CH, D), jnp.float32),
            pltpu.VMEM((ZR, D), jnp.float32),
            pltpu.VMEM_SHARED((NP, D), jnp.float32),
        ] + [pltpu.SemaphoreType.DMA] * 16)


def _dense_body(relu, h_ref, aggp_ref, degp_ref, ws_ref, wn_ref, b_ref, o_ref):
    p = aggp_ref[0] + aggp_ref[1]
    deg = degp_ref[0, :, 0:1] + degp_ref[1, :, 0:1]
    hn = p * (1.0 / jnp.maximum(deg, 1.0))
    acc = jnp.dot(h_ref[...], ws_ref[...], preferred_element_type=jnp.float32)
    acc = acc + jnp.dot(hn, wn_ref[...], preferred_element_type=jnp.float32)
    acc = acc + b_ref[...]
    if relu:
        acc = jnp.maximum(acc, 0.0)
    o_ref[...] = acc


BLK = 1000


def _dense(h, aggp, degp, w_self, w_neigh, b, relu):
    return pl.pallas_call(
        functools.partial(_dense_body, relu),
        grid=(N // BLK,),
        in_specs=[
            pl.BlockSpec((BLK, D), lambda i: (i, 0)),
            pl.BlockSpec((NC, BLK, D), lambda i: (0, i, 0)),
            pl.BlockSpec((NC, BLK, D), lambda i: (0, i, 0)),
            pl.BlockSpec((D, D), lambda i: (0, 0)),
            pl.BlockSpec((D, D), lambda i: (0, 0)),
            pl.BlockSpec((1, D), lambda i: (0, 0)),
        ],
        out_specs=pl.BlockSpec((BLK, D), lambda i: (i, 0)),
        out_shape=jax.ShapeDtypeStruct((N, D), jnp.float32),
    )(h, aggp, degp, w_self, w_neigh, b)


def kernel(x, edge_index, W1_self, W1_neigh, b1, W2_self, W2_neigh, b2):
    srcf = edge_index[0]
    dstf = edge_index[1]
    b1r = b1.reshape(1, D)
    b2r = b2.reshape(1, D)

    agg1p, degp = _make_sc_agg(True)(x, srcf, dstf)
    h1 = _dense(x, agg1p, degp, W1_self, W1_neigh, b1r, relu=True)
    agg2p = _make_sc_agg(False)(h1, srcf, dstf)
    out = _dense(h1, agg2p, degp, W2_self, W2_neigh, b2r, relu=False)
    return out


# speedup vs baseline: 1.7614x; 1.0324x over previous
import functools

import jax
import jax.numpy as jnp
from jax import lax
from jax.experimental import pallas as pl
from jax.experimental.pallas import tpu as pltpu
from jax.experimental.pallas import tpu_sc as plsc

N = 10000
NP = 10240
E = 320000
D = 128
NC = 2
NS = 16
NW = NC * NS
E_PER_W = E // NW
CH = 80
NCH = E_PER_W // CH
RPS = NP // NS
ZR = 16

_MESH = dict(core_axis_name="c", subcore_axis_name="s",
             num_cores=NC, num_subcores=NS)


def _fill(ref, rows, value):
    @pl.loop(0, rows)
    def _(r):
        @pl.loop(0, D // 16)
        def _(c):
            ref[r, pl.ds(c * 16, 16)] = jnp.full((16,), value, jnp.float32)


def _zero_accum(sid, zrow_v, accum_sh):
    @pl.loop(0, RPS // ZR)
    def _(j):
        pltpu.sync_copy(zrow_v, accum_sh.at[pl.ds(sid * RPS + j * ZR, ZR), :])


def _sc_agg_body(with_deg, h_hbm, srcf_hbm, dstf_hbm, *refs):
    if with_deg:
        (out_hbm, deg_out,
         sidx_v, didx_v, rows0_v, rows1_v, rows2_v, rows3_v,
         zrow_v, accum_sh, *sems) = refs
    else:
        (out_hbm,
         sidx_v, didx_v, rows0_v, rows1_v, rows2_v, rows3_v,
         zrow_v, accum_sh, *sems) = refs
    cid = lax.axis_index("c")
    sid = lax.axis_index("s")
    wid = sid * NC + cid
    ebase = wid * E_PER_W
    rs = pl.ds(sid * RPS, RPS)

    isems = sems[0:8]
    rsems = sems[8:12]
    ssems = sems[12:16]
    rows = (rows0_v, rows1_v, rows2_v, rows3_v)

    _fill(zrow_v, ZR, 0.0)
    _zero_accum(sid, zrow_v, accum_sh)
    if with_deg:
        _fill(rows0_v, CH, 1.0)
    plsc.subcore_barrier()

    def idx_load(m, s):
        off = pl.multiple_of(ebase + m * CH, 8)
        pltpu.async_copy(srcf_hbm.at[pl.ds(off, CH)], sidx_v.at[s], isems[s])
        pltpu.async_copy(dstf_hbm.at[pl.ds(off, CH)], didx_v.at[s], isems[s])

    def idx_wait(s):
        pltpu.make_async_copy(srcf_hbm.at[pl.ds(0, CH)], sidx_v.at[s],
                              isems[s]).wait()
        pltpu.make_async_copy(dstf_hbm.at[pl.ds(0, CH)], didx_v.at[s],
                              isems[s]).wait()

    def gather_start(s, r):
        pltpu.async_copy(h_hbm.at[sidx_v.at[s]], rows[r], rsems[r])

    def gather_wait(r):
        pltpu.make_async_copy(h_hbm.at[sidx_v.at[0]], rows[r], rsems[r]).wait()

    def scatter_start(s, r, buf=None):
        pltpu.async_copy(rows[r if buf is None else buf],
                         accum_sh.at[didx_v.at[s]], ssems[r], add=True)

    def scatter_wait(r):
        pltpu.make_async_copy(rows[0], accum_sh.at[didx_v.at[0]],
                              ssems[r]).wait()

    if with_deg:
        def deg_body(m, b8, dyn):
            b4 = b8 % 4
            if dyn:
                scatter_wait(b4)

                @pl.when(m + 4 < NCH)
                def _():
                    idx_load(m + 4, (b8 + 4) % 8)
            else:
                if m >= 4:
                    scatter_wait(b4)
                    if m + 4 < NCH:
                        idx_load(m + 4, (b8 + 4) % 8)
            idx_wait(b8)
            scatter_start(b8, b4, buf=0)

        for s in range(8):
            idx_load(s, s)
        for m in range(8):
            deg_body(m, m % 8, False)

        @pl.loop(1, NCH // 8)
        def _(q):
            for b in range(8):
                deg_body(q * 8 + b, b, True)

        for m in range((NCH // 8) * 8, NCH):
            deg_body(m, m % 8, False)
        for k in range(NCH - 4, NCH):
            scatter_wait(k % 4)

        plsc.subcore_barrier()
        pltpu.sync_copy(accum_sh.at[rs, :], deg_out.at[cid, rs, :])
        plsc.subcore_barrier()

    def agg_body(m, b8, dyn):
        b4 = b8 % 4
        gather_wait(b4)
        scatter_start(b8, b4)
        if dyn or m >= 2:
            scatter_wait((b4 + 2) % 4)
            if dyn:
                @pl.when(m + 6 < NCH)
                def _():
                    idx_load(m + 6, (b8 + 6) % 8)
            elif m + 6 < NCH:
                idx_load(m + 6, (b8 + 6) % 8)
        if dyn:
            @pl.when(m + 2 < NCH)
            def _():
                idx_wait((b8 + 2) % 8)
                gather_start((b8 + 2) % 8, (b4 + 2) % 4)
        elif m + 2 < NCH:
            idx_wait((b8 + 2) % 8)
            gather_start((b8 + 2) % 8, (b4 + 2) % 4)

    for s in range(8):
        idx_load(s, s)
    idx_wait(0)
    gather_start(0, 0)
    idx_wait(1)
    gather_start(1, 1)

    for m in range(8):
        agg_body(m, m % 8, False)

    @pl.loop(1, NCH // 8)
    def _(q):
        for b in range(8):
            agg_body(q * 8 + b, b, True)

    for m in range((NCH // 8) * 8, NCH):
        agg_body(m, m % 8, False)

    scatter_wait((NCH - 2) % 4)
    scatter_wait((NCH - 1) % 4)

    plsc.subcore_barrier()

    pltpu.sync_copy(accum_sh.at[rs, :], out_hbm.at[cid, rs, :])


def _make_sc_agg(with_deg):
    out_type = jax.ShapeDtypeStruct((NC, NP, D), jnp.float32)
    return pl.kernel(
        functools.partial(_sc_agg_body, with_deg),
        out_type=(out_type, out_type) if with_deg else out_type,
        mesh=plsc.VectorSubcoreMesh(**_MESH),
        scratch_types=[
            pltpu.VMEM((8, CH), jnp.int32),
            pltpu.VMEM((8, CH), jnp.int32),
            pltpu.VMEM((CH, D), jnp.float32),
            pltpu.VMEM((CH, D), jnp.float32),
            pltpu.VMEM((CH, D), jnp.float32),
            pltpu.VMEM((CH, D), jnp.float32),
            pltpu.VMEM((ZR, D), jnp.float32),
            pltpu.VMEM_SHARED((NP, D), jnp.float32),
        ] + [pltpu.SemaphoreType.DMA] * 16)


def _dense_body(relu, sub_deg, h_ref, aggp_ref, degp_ref, ws_ref, wn_ref,
                b_ref, o_ref):
    p = aggp_ref[0] + aggp_ref[1]
    degf = degp_ref[0] + degp_ref[1]
    if sub_deg:
        p = p - degf
    deg = degf[:, 0:1]
    hn = p * (1.0 / jnp.maximum(deg, 1.0))
    acc = jnp.dot(h_ref[...], ws_ref[...], preferred_element_type=jnp.float32)
    acc = acc + jnp.dot(hn, wn_ref[...], preferred_element_type=jnp.float32)
    acc = acc + b_ref[...]
    if relu:
        acc = jnp.maximum(acc, 0.0)
    o_ref[...] = acc


BLK = 1000


def _dense(h, aggp, degp, w_self, w_neigh, b, relu, sub_deg):
    return pl.pallas_call(
        functools.partial(_dense_body, relu, sub_deg),
        grid=(N // BLK,),
        in_specs=[
            pl.BlockSpec((BLK, D), lambda i: (i, 0)),
            pl.BlockSpec((NC, BLK, D), lambda i: (0, i, 0)),
            pl.BlockSpec((NC, BLK, D), lambda i: (0, i, 0)),
            pl.BlockSpec((D, D), lambda i: (0, 0)),
            pl.BlockSpec((D, D), lambda i: (0, 0)),
            pl.BlockSpec((1, D), lambda i: (0, 0)),
        ],
        out_specs=pl.BlockSpec((BLK, D), lambda i: (i, 0)),
        out_shape=jax.ShapeDtypeStruct((N, D), jnp.float32),
    )(h, aggp, degp, w_self, w_neigh, b)


def kernel(x, edge_index, W1_self, W1_neigh, b1, W2_self, W2_neigh, b2):
    srcf = edge_index[0]
    dstf = edge_index[1]
    b1r = b1.reshape(1, D)
    b2r = b2.reshape(1, D)

    agg1p, degp = _make_sc_agg(True)(x, srcf, dstf)
    h1 = _dense(x, agg1p, degp, W1_self, W1_neigh, b1r, relu=True,
                sub_deg=True)
    agg2p = _make_sc_agg(False)(h1, srcf, dstf)
    out = _dense(h1, agg2p, degp, W2_self, W2_neigh, b2r, relu=False,
                 sub_deg=False)
    return out


# speedup vs baseline: 1.7676x; 1.0035x over previous
import functools

import jax
import jax.numpy as jnp
from jax import lax
from jax.experimental import pallas as pl
from jax.experimental.pallas import tpu as pltpu
from jax.experimental.pallas import tpu_sc as plsc

N = 10000
NP = 10240
E = 320000
D = 128
NC = 2
NS = 16
NW = NC * NS
E_PER_W = E // NW
CH = 80
NCH = E_PER_W // CH
RPS = NP // NS
ZR = 16

_MESH = dict(core_axis_name="c", subcore_axis_name="s",
             num_cores=NC, num_subcores=NS)


def _fill(ref, rows, value):
    @pl.loop(0, rows)
    def _(r):
        @pl.loop(0, D // 16)
        def _(c):
            ref[r, pl.ds(c * 16, 16)] = jnp.full((16,), value, jnp.float32)


def _zero_accum(sid, zrow_v, accum_sh):
    @pl.loop(0, RPS // ZR)
    def _(j):
        pltpu.sync_copy(zrow_v, accum_sh.at[pl.ds(sid * RPS + j * ZR, ZR), :])


def _sc_agg_body(with_deg, h_hbm, srcf_hbm, dstf_hbm, *refs):
    if with_deg:
        (out_hbm, deg_out,
         sidx_v, didx_v, rows0_v, rows1_v, rows2_v, rows3_v,
         zrow_v, accum_sh, *sems) = refs
    else:
        (out_hbm,
         sidx_v, didx_v, rows0_v, rows1_v, rows2_v, rows3_v,
         zrow_v, accum_sh, *sems) = refs
    cid = lax.axis_index("c")
    sid = lax.axis_index("s")
    wid = sid * NC + cid
    ebase = wid * E_PER_W
    rs = pl.ds(sid * RPS, RPS)

    isems = sems[0:8]
    rsems = sems[8:12]
    ssems = sems[12:16]
    rows = (rows0_v, rows1_v, rows2_v, rows3_v)

    _fill(zrow_v, ZR, 0.0)
    _zero_accum(sid, zrow_v, accum_sh)
    if with_deg:
        _fill(rows0_v, CH, 1.0)
    plsc.subcore_barrier()

    def idx_load(m, s):
        off = pl.multiple_of(ebase + m * CH, 8)
        pltpu.async_copy(srcf_hbm.at[pl.ds(off, CH)], sidx_v.at[s], isems[s])
        pltpu.async_copy(dstf_hbm.at[pl.ds(off, CH)], didx_v.at[s], isems[s])

    def idx_wait(s):
        pltpu.make_async_copy(srcf_hbm.at[pl.ds(0, CH)], sidx_v.at[s],
                              isems[s]).wait()
        pltpu.make_async_copy(dstf_hbm.at[pl.ds(0, CH)], didx_v.at[s],
                              isems[s]).wait()

    def gather_start(s, r):
        pltpu.async_copy(h_hbm.at[sidx_v.at[s]], rows[r], rsems[r])

    def gather_wait(r):
        pltpu.make_async_copy(h_hbm.at[sidx_v.at[0]], rows[r], rsems[r]).wait()

    def scatter_start(s, r, buf=None):
        pltpu.async_copy(rows[r if buf is None else buf],
                         accum_sh.at[didx_v.at[s]], ssems[r], add=True)

    def scatter_wait(r):
        pltpu.make_async_copy(rows[0], accum_sh.at[didx_v.at[0]],
                              ssems[r]).wait()

    if with_deg:
        def deg_body(m, b8, dyn):
            b4 = b8 % 4
            if dyn:
                scatter_wait(b4)

                @pl.when(m + 4 < NCH)
                def _():
                    idx_load(m + 4, (b8 + 4) % 8)
            else:
                if m >= 4:
                    scatter_wait(b4)
                    if m + 4 < NCH:
                        idx_load(m + 4, (b8 + 4) % 8)
            idx_wait(b8)
            scatter_start(b8, b4, buf=0)

        for s in range(8):
            idx_load(s, s)
        for m in range(8):
            deg_body(m, m % 8, False)

        @pl.loop(1, NCH // 8)
        def _(q):
            for b in range(8):
                deg_body(q * 8 + b, b, True)

        for m in range((NCH // 8) * 8, NCH):
            deg_body(m, m % 8, False)
        for k in range(NCH - 4, NCH):
            scatter_wait(k % 4)

        plsc.subcore_barrier()
        pltpu.sync_copy(accum_sh.at[rs, :], deg_out.at[cid, rs, :])
        plsc.subcore_barrier()

    def agg_body(m, b8, dyn):
        b4 = b8 % 4
        gather_wait(b4)
        scatter_start(b8, b4)
        if dyn or m >= 2:
            scatter_wait((b4 + 2) % 4)
            if dyn:
                @pl.when(m + 6 < NCH)
                def _():
                    idx_load(m + 6, (b8 + 6) % 8)
            elif m + 6 < NCH:
                idx_load(m + 6, (b8 + 6) % 8)
        if dyn:
            @pl.when(m + 2 < NCH)
            def _():
                idx_wait((b8 + 2) % 8)
                gather_start((b8 + 2) % 8, (b4 + 2) % 4)
        elif m + 2 < NCH:
            idx_wait((b8 + 2) % 8)
            gather_start((b8 + 2) % 8, (b4 + 2) % 4)

    for s in range(8):
        idx_load(s, s)
    idx_wait(0)
    gather_start(0, 0)
    idx_wait(1)
    gather_start(1, 1)

    for m in range(8):
        agg_body(m, m % 8, False)

    @pl.loop(1, NCH // 8)
    def _(q):
        for b in range(8):
            agg_body(q * 8 + b, b, True)

    for m in range((NCH // 8) * 8, NCH):
        agg_body(m, m % 8, False)

    scatter_wait((NCH - 2) % 4)
    scatter_wait((NCH - 1) % 4)

    plsc.subcore_barrier()

    pltpu.sync_copy(accum_sh.at[rs, :], out_hbm.at[cid, rs, :])


def _make_sc_agg(with_deg):
    out_type = jax.ShapeDtypeStruct((NC, NP, D), jnp.float32)
    return pl.kernel(
        functools.partial(_sc_agg_body, with_deg),
        out_type=(out_type, out_type) if with_deg else out_type,
        mesh=plsc.VectorSubcoreMesh(**_MESH),
        scratch_types=[
            pltpu.VMEM((8, CH), jnp.int32),
            pltpu.VMEM((8, CH), jnp.int32),
            pltpu.VMEM((CH, D), jnp.float32),
            pltpu.VMEM((CH, D), jnp.float32),
            pltpu.VMEM((CH, D), jnp.float32),
            pltpu.VMEM((CH, D), jnp.float32),
            pltpu.VMEM((ZR, D), jnp.float32),
            pltpu.VMEM_SHARED((NP, D), jnp.float32),
        ] + [pltpu.SemaphoreType.DMA] * 16)


BLK = 1000


def _selfmm_body(h_ref, w_ref, b_ref, o_ref):
    o_ref[...] = jnp.dot(h_ref[...], w_ref[...],
                         preferred_element_type=jnp.float32) + b_ref[...]


def _selfmm(h, w, b):
    return pl.pallas_call(
        _selfmm_body,
        grid=(N // BLK,),
        in_specs=[
            pl.BlockSpec((BLK, D), lambda i: (i, 0)),
            pl.BlockSpec((D, D), lambda i: (0, 0)),
            pl.BlockSpec((1, D), lambda i: (0, 0)),
        ],
        out_specs=pl.BlockSpec((BLK, D), lambda i: (i, 0)),
        out_shape=jax.ShapeDtypeStruct((N, D), jnp.float32),
    )(h, w, b)


def _finish_body(relu, sub_deg, s_ref, aggp_ref, degp_ref, wn_ref, o_ref):
    p = aggp_ref[0] + aggp_ref[1]
    degf = degp_ref[0] + degp_ref[1]
    if sub_deg:
        p = p - degf
    hn = p * (1.0 / jnp.maximum(degf[:, 0:1], 1.0))
    acc = s_ref[...] + jnp.dot(hn, wn_ref[...],
                               preferred_element_type=jnp.float32)
    if relu:
        acc = jnp.maximum(acc, 0.0)
    o_ref[...] = acc


def _finish(s, aggp, degp, w_neigh, relu, sub_deg):
    return pl.pallas_call(
        functools.partial(_finish_body, relu, sub_deg),
        grid=(N // BLK,),
        in_specs=[
            pl.BlockSpec((BLK, D), lambda i: (i, 0)),
            pl.BlockSpec((NC, BLK, D), lambda i: (0, i, 0)),
            pl.BlockSpec((NC, BLK, D), lambda i: (0, i, 0)),
            pl.BlockSpec((D, D), lambda i: (0, 0)),
        ],
        out_specs=pl.BlockSpec((BLK, D), lambda i: (i, 0)),
        out_shape=jax.ShapeDtypeStruct((N, D), jnp.float32),
    )(s, aggp, degp, w_neigh)


def kernel(x, edge_index, W1_self, W1_neigh, b1, W2_self, W2_neigh, b2):
    srcf = edge_index[0]
    dstf = edge_index[1]
    b1r = b1.reshape(1, D)
    b2r = b2.reshape(1, D)

    s1 = _selfmm(x, W1_self, b1r)
    agg1p, degp = _make_sc_agg(True)(x, srcf, dstf)
    h1 = _finish(s1, agg1p, degp, W1_neigh, relu=True, sub_deg=True)
    s2 = _selfmm(h1, W2_self, b2r)
    agg2p = _make_sc_agg(False)(h1, srcf, dstf)
    out = _finish(s2, agg2p, degp, W2_neigh, relu=False, sub_deg=False)
    return out
